# Initial kernel scaffold; baseline (speedup 1.0000x reference)
import jax
import jax.numpy as jnp
from jax.experimental import pallas as pl


def kernel(x, W, b, predicate_matrix):
    raise NotImplementedError("write your pallas kernel here")


# speedup vs baseline: 10.9605x; 10.9605x over previous
import jax
import jax.numpy as jnp
from jax.experimental import pallas as pl


def _fused_body(x_ref, w_ref, b_ref, pt_ref, o_ref):
    feats = jnp.dot(
        x_ref[...].astype(jnp.bfloat16),
        w_ref[...].astype(jnp.bfloat16),
        preferred_element_type=jnp.float32,
    )
    feats = feats + b_ref[...]
    f = (feats > 0.5).astype(jnp.bfloat16)
    pm1 = pt_ref[...].astype(jnp.bfloat16) - jnp.bfloat16(1.0)
    o_ref[...] = jnp.dot(f, pm1, preferred_element_type=jnp.float32)


def kernel(x, W, b, predicate_matrix):
    B, D = x.shape
    F = W.shape[1]
    C = predicate_matrix.shape[0]
    bm = 256 if B % 256 == 0 else B
    pt = predicate_matrix.T
    b2 = b.reshape(1, F)
    return pl.pallas_call(
        _fused_body,
        grid=(B // bm,),
        in_specs=[
            pl.BlockSpec((bm, D), lambda i: (i, 0)),
            pl.BlockSpec((D, F), lambda i: (0, 0)),
            pl.BlockSpec((1, F), lambda i: (0, 0)),
            pl.BlockSpec((F, C), lambda i: (0, 0)),
        ],
        out_specs=pl.BlockSpec((bm, C), lambda i: (i, 0)),
        out_shape=jax.ShapeDtypeStruct((B, C), jnp.float32),
    )(x, W, b2, pt)
